# baseline (speedup 1.0000x reference)
import functools

import jax
import jax.numpy as jnp
from jax import lax
from jax.experimental import pallas as pl
from jax.experimental.pallas import tpu as pltpu
from jax.experimental.pallas import tpu_sc as plsc

NC = 2
NS = 16
LANES = 16
NW = NC * NS


@functools.partial(jax.jit, static_argnums=(2, 3, 4, 5))
def _embedding_bag_mean(x_flat, wrep_flat, B, L, E, D):
    chunk = B // NW
    nfull = L // LANES
    rem = L % LANES

    mesh = plsc.VectorSubcoreMesh(core_axis_name="c", subcore_axis_name="s")

    @functools.partial(
        pl.kernel,
        out_type=jax.ShapeDtypeStruct((B * D,), jnp.float32),
        mesh=mesh,
        scratch_types=[
            pltpu.VMEM((chunk * L,), jnp.int32),
            pltpu.VMEM((chunk * D,), jnp.float32),
            pltpu.VMEM((E * LANES,), jnp.float32),
            pltpu.VMEM((LANES,), jnp.float32),
        ],
        compiler_params=pltpu.CompilerParams(needs_layout_passes=False),
    )
    def sc_kernel(x_hbm, wrep_hbm, out_hbm, x_v, out_v, w_v, cnt_v):
        wid = lax.axis_index("s") * NC + lax.axis_index("c")
        base = wid * chunk
        pltpu.sync_copy(wrep_hbm, w_v)
        pltpu.sync_copy(x_hbm.at[pl.ds(base * L, chunk * L)], x_v)

        lane = lax.iota(jnp.int32, LANES)
        zero16 = jnp.zeros((LANES,), jnp.float32)
        ones16 = jnp.ones((LANES,), jnp.float32)
        out_mask = lane < D
        tail_mask = lane >= (LANES - rem)
        wrows = [w_v[pl.ds(e * LANES, LANES)] for e in range(E)]

        def bag_body(bag, _):
            cnt_v[...] = zero16
            xbase = bag * L
            for i in range(nfull):
                xv = x_v[pl.ds(xbase + i * LANES, LANES)]
                plsc.addupdate_scatter(cnt_v, [xv], ones16)
            if rem:
                xv = x_v[pl.ds(xbase + L - LANES, LANES)]
                plsc.addupdate_scatter(cnt_v, [xv], ones16, mask=tail_mask)

            cnt = cnt_v[...]
            acc = zero16
            for e in range(E):
                acc = acc + wrows[e] * cnt[e]
            plsc.store_scatter(out_v, [bag * D + lane], acc, mask=out_mask)
            return 0

        lax.fori_loop(0, chunk, bag_body, 0)
        pltpu.sync_copy(out_v, out_hbm.at[pl.ds(base * D, chunk * D)])

    return sc_kernel(x_flat, wrep_flat)


def kernel(x, weight):
    B, L = x.shape
    E, D = weight.shape
    x_flat = x.astype(jnp.int32).reshape(-1)
    wrep = (
        jnp.zeros((E, LANES), jnp.float32)
        .at[:, :D].set(weight.astype(jnp.float32) * (1.0 / L))
        .reshape(-1)
    )
    out = _embedding_bag_mean(x_flat, wrep, B, L, E, D)
    return out.reshape(B, D)


# speedup vs baseline: 1.2412x; 1.2412x over previous
import functools

import jax
import jax.numpy as jnp
from jax import lax
from jax.experimental import pallas as pl
from jax.experimental.pallas import tpu as pltpu
from jax.experimental.pallas import tpu_sc as plsc

NC = 2
NS = 16
LANES = 16
NW = NC * NS
UNROLL = 3


def _align8(n):
    return ((n + 7) // 8) * 8


def _table_layout(E):
    t3 = E * E * E
    poff = _align8(t3)
    soff = _align8(poff + E * E)
    stride = _align8(soff + E)
    return poff, soff, stride


@functools.partial(jax.jit, static_argnums=(2, 3, 4, 5))
def _embedding_bag_mean(x_flat, tbl_flat, B, L, E, D):
    chunk = B // NW
    groups = chunk // LANES
    ntrip = L // 3
    npair = (L - 3 * ntrip) // 2
    nsing = L - 3 * ntrip - 2 * npair
    nfull = ntrip // UNROLL
    POFF, SOFF, STRIDE = _table_layout(E)

    mesh = plsc.VectorSubcoreMesh(core_axis_name="c", subcore_axis_name="s")

    @functools.partial(
        pl.kernel,
        out_type=jax.ShapeDtypeStruct((B * D,), jnp.float32),
        mesh=mesh,
        scratch_types=[
            pltpu.VMEM((chunk * L,), jnp.int32),
            pltpu.VMEM((chunk * D,), jnp.float32),
            pltpu.VMEM((D * STRIDE,), jnp.float32),
        ],
        compiler_params=pltpu.CompilerParams(needs_layout_passes=False),
    )
    def sc_kernel(x_hbm, tbl_hbm, out_hbm, x_v, out_v, tbl_v):
        wid = lax.axis_index("s") * NC + lax.axis_index("c")
        base = wid * chunk
        pltpu.sync_copy(tbl_hbm, tbl_v)
        pltpu.sync_copy(x_hbm.at[pl.ds(base * L, chunk * L)], x_v)

        tsub = [tbl_v.at[pl.ds(d * STRIDE, E * E * E)] for d in range(D)]
        psub = [tbl_v.at[pl.ds(d * STRIDE + POFF, E * E)] for d in range(D)]
        ssub = [tbl_v.at[pl.ds(d * STRIDE + SOFF, E)] for d in range(D)]

        lane = lax.iota(jnp.int32, LANES)
        scale = jnp.float32(1.0 / L)
        e_vec = jnp.full((LANES,), E, jnp.int32)

        def trip_idx(pos):
            xv1 = plsc.load_gather(x_v, [pos])
            xv2 = plsc.load_gather(x_v, [pos + 1])
            xv3 = plsc.load_gather(x_v, [pos + 2])
            return (xv1 * e_vec + xv2) * e_vec + xv3

        def group_body(g, _):
            rows = g * LANES + lane
            flat_base = rows * L

            def trip_body(t, accs):
                accs = list(accs)
                pos0 = flat_base + 3 * t * UNROLL
                for u in range(UNROLL):
                    tidx = trip_idx(pos0 + 3 * u)
                    for d in range(D):
                        accs[d] = accs[d] + plsc.load_gather(tsub[d], [tidx])
                return tuple(accs)

            accs = lax.fori_loop(
                0, nfull, trip_body,
                tuple(jnp.zeros((LANES,), jnp.float32) for _ in range(D)),
            )
            accs = list(accs)
            for t in range(nfull * UNROLL, ntrip):
                tidx = trip_idx(flat_base + 3 * t)
                for d in range(D):
                    accs[d] = accs[d] + plsc.load_gather(tsub[d], [tidx])
            for p in range(npair):
                pos = flat_base + 3 * ntrip + 2 * p
                xv1 = plsc.load_gather(x_v, [pos])
                xv2 = plsc.load_gather(x_v, [pos + 1])
                pidx = xv1 * e_vec + xv2
                for d in range(D):
                    accs[d] = accs[d] + plsc.load_gather(psub[d], [pidx])
            for s in range(nsing):
                xv = plsc.load_gather(x_v, [flat_base + (L - 1)])
                for d in range(D):
                    accs[d] = accs[d] + plsc.load_gather(ssub[d], [xv])

            out_base = rows * D
            for d in range(D):
                plsc.store_scatter(out_v, [out_base + d], accs[d] * scale)
            return 0

        lax.fori_loop(0, groups, group_body, 0)
        pltpu.sync_copy(out_v, out_hbm.at[pl.ds(base * D, chunk * D)])

    return sc_kernel(x_flat, tbl_flat)


def kernel(x, weight):
    B, L = x.shape
    E, D = weight.shape
    x_flat = x.astype(jnp.int32).reshape(-1)
    w = weight.astype(jnp.float32)
    poff, soff, stride = _table_layout(E)
    pairs = (w[:, None, :] + w[None, :, :]).reshape(E * E, D)
    trips = (pairs[:, None, :] + w[None, :, :]).reshape(E * E * E, D)
    tbl = (
        jnp.zeros((D, stride), jnp.float32)
        .at[:, : E * E * E].set(trips.T)
        .at[:, poff: poff + E * E].set(pairs.T)
        .at[:, soff: soff + E].set(w.T)
        .reshape(-1)
    )
    out = _embedding_bag_mean(x_flat, tbl, B, L, E, D)
    return out.reshape(B, D)
